# baseline (device time: 62455 ns/iter reference)
import jax
import jax.numpy as jnp
from jax import lax
from jax.experimental import pallas as pl
from jax.experimental.pallas import tpu as pltpu

N_DEV = 8


def kernel(dy, W):
    m, _ = dy.shape
    d = W.shape[0]

    def body(dy_ref, w_ref, out_ref, comm_ref, send_sems, recv_sems):
        my = lax.axis_index("i")
        left = (my - 1) % N_DEV
        right = (my + 1) % N_DEV

        barrier_sem = pltpu.get_barrier_semaphore()
        for nbr in (left, right):
            pl.semaphore_signal(
                barrier_sem, inc=1,
                device_id=(nbr,), device_id_type=pl.DeviceIdType.MESH,
            )
        pl.semaphore_wait(barrier_sem, 2)

        partial = lax.dot_general(
            dy_ref[:, :], w_ref[:, :],
            (((1,), (1,)), ((), ())),
            preferred_element_type=jnp.float32,
        )
        out_ref[:, :] = partial
        comm_ref[0, :, :] = partial.astype(jnp.bfloat16)

        for h in range(N_DEV - 1):
            rdma = pltpu.make_async_remote_copy(
                src_ref=comm_ref.at[h],
                dst_ref=comm_ref.at[h + 1],
                send_sem=send_sems.at[h],
                recv_sem=recv_sems.at[h],
                device_id=(right,),
                device_id_type=pl.DeviceIdType.MESH,
            )
            rdma.start()
            rdma.wait()
            out_ref[:, :] = out_ref[:, :] + comm_ref[h + 1, :, :].astype(
                jnp.float32
            )

    return pl.pallas_call(
        body,
        out_shape=jax.ShapeDtypeStruct((m, d), jnp.float32),
        in_specs=[
            pl.BlockSpec(memory_space=pltpu.VMEM),
            pl.BlockSpec(memory_space=pltpu.VMEM),
        ],
        out_specs=pl.BlockSpec(memory_space=pltpu.VMEM),
        scratch_shapes=[
            pltpu.VMEM((N_DEV, m, d), jnp.bfloat16),
            pltpu.SemaphoreType.DMA((N_DEV - 1,)),
            pltpu.SemaphoreType.DMA((N_DEV - 1,)),
        ],
        compiler_params=pltpu.CompilerParams(collective_id=0),
    )(dy.astype(jnp.bfloat16), W.astype(jnp.bfloat16))


# device time: 21560 ns/iter; 2.8968x vs baseline; 2.8968x over previous
import jax
import jax.numpy as jnp
from jax import lax
from jax.experimental import pallas as pl
from jax.experimental.pallas import tpu as pltpu

N_DEV = 8
CHUNK = 512 // N_DEV


def kernel(dy, W):
    m, _ = dy.shape
    d = W.shape[0]

    def body(dy_ref, w_ref, out_ref, part_ref, red_ref, rs_buf, ag_buf,
             rs_send, rs_recv, ag_send, ag_recv):
        my = lax.axis_index("i")

        barrier_sem = pltpu.get_barrier_semaphore()
        for r in range(1, N_DEV):
            pl.semaphore_signal(
                barrier_sem, inc=1,
                device_id=((my + r) % N_DEV,),
                device_id_type=pl.DeviceIdType.MESH,
            )
        pl.semaphore_wait(barrier_sem, N_DEV - 1)

        partial = lax.dot_general(
            dy_ref[:, :], w_ref[:, :],
            (((1,), (1,)), ((), ())),
            preferred_element_type=jnp.float32,
        )
        part_ref[:, :, :] = partial.astype(jnp.bfloat16).reshape(
            N_DEV, CHUNK, d
        )
        out_ref[:, :] = partial

        rs_rdmas = []
        for r in range(1, N_DEV):
            dst = (my + r) % N_DEV
            rdma = pltpu.make_async_remote_copy(
                src_ref=part_ref.at[dst],
                dst_ref=rs_buf.at[N_DEV - 1 - r],
                send_sem=rs_send.at[r - 1],
                recv_sem=rs_recv.at[N_DEV - 1 - r],
                device_id=(dst,),
                device_id_type=pl.DeviceIdType.MESH,
            )
            rdma.start()
            rs_rdmas.append(rdma)

        for s in range(N_DEV - 1):
            pltpu.make_async_remote_copy(
                src_ref=rs_buf.at[s], dst_ref=rs_buf.at[s],
                send_sem=rs_send.at[s], recv_sem=rs_recv.at[s],
                device_id=(my,), device_id_type=pl.DeviceIdType.MESH,
            ).wait_recv()

        red = out_ref[pl.ds(my * CHUNK, CHUNK), :]
        for s in range(N_DEV - 1):
            red = red + rs_buf[s, :, :].astype(jnp.float32)
        red_ref[:, :] = red.astype(jnp.bfloat16)
        out_ref[pl.ds(my * CHUNK, CHUNK), :] = red

        ag_rdmas = []
        for r in range(1, N_DEV):
            dst = (my + r) % N_DEV
            rdma = pltpu.make_async_remote_copy(
                src_ref=red_ref,
                dst_ref=ag_buf.at[N_DEV - 1 - r],
                send_sem=ag_send.at[r - 1],
                recv_sem=ag_recv.at[N_DEV - 1 - r],
                device_id=(dst,),
                device_id_type=pl.DeviceIdType.MESH,
            )
            rdma.start()
            ag_rdmas.append(rdma)

        for s in range(N_DEV - 1):
            pltpu.make_async_remote_copy(
                src_ref=ag_buf.at[s], dst_ref=ag_buf.at[s],
                send_sem=ag_send.at[s], recv_sem=ag_recv.at[s],
                device_id=(my,), device_id_type=pl.DeviceIdType.MESH,
            ).wait_recv()
            origin = (my + s + 1) % N_DEV
            out_ref[pl.ds(origin * CHUNK, CHUNK), :] = ag_buf[
                s, :, :
            ].astype(jnp.float32)

        for rdma in rs_rdmas + ag_rdmas:
            rdma.wait_send()

    return pl.pallas_call(
        body,
        out_shape=jax.ShapeDtypeStruct((m, d), jnp.float32),
        in_specs=[
            pl.BlockSpec(memory_space=pltpu.VMEM),
            pl.BlockSpec(memory_space=pltpu.VMEM),
        ],
        out_specs=pl.BlockSpec(memory_space=pltpu.VMEM),
        scratch_shapes=[
            pltpu.VMEM((N_DEV, CHUNK, d), jnp.bfloat16),
            pltpu.VMEM((CHUNK, d), jnp.bfloat16),
            pltpu.VMEM((N_DEV - 1, CHUNK, d), jnp.bfloat16),
            pltpu.VMEM((N_DEV - 1, CHUNK, d), jnp.bfloat16),
            pltpu.SemaphoreType.DMA((N_DEV - 1,)),
            pltpu.SemaphoreType.DMA((N_DEV - 1,)),
            pltpu.SemaphoreType.DMA((N_DEV - 1,)),
            pltpu.SemaphoreType.DMA((N_DEV - 1,)),
        ],
        compiler_params=pltpu.CompilerParams(collective_id=0),
    )(dy.astype(jnp.bfloat16), W.astype(jnp.bfloat16))
